# baseline (speedup 1.0000x reference)
import functools

import jax
import jax.numpy as jnp
from jax import lax
from jax.experimental import pallas as pl
from jax.experimental.pallas import tpu as pltpu
from jax.experimental.pallas import tpu_sc as plsc

BATCH = 4096
HIST = 20
DIM = 64
NUM_ROWS = BATCH * HIST
NPAIR = 500000
NC, NS = 2, 16
NW = NC * NS
RPW = NUM_ROWS // NW
CHUNK = 128
NCHUNK = RPW // CHUNK
L = 16

_mesh = plsc.VectorSubcoreMesh(core_axis_name="c", subcore_axis_name="s")

_TBLK = 512
_TGRID = 977
SPLIT = _TGRID * _TBLK


def _transpose_body(in0_ref, in1_ref, out_ref):
    out_ref[:, 0:DIM] = in0_ref[...].T
    out_ref[:, DIM:128] = in1_ref[...].T


_tc_transpose = pl.pallas_call(
    _transpose_body,
    grid=(_TGRID,),
    in_specs=[
        pl.BlockSpec((DIM, _TBLK), lambda j: (0, j)),
        pl.BlockSpec((DIM, _TBLK), lambda j: (0, j + _TGRID)),
    ],
    out_specs=pl.BlockSpec((_TBLK, 128), lambda j: (j, 0)),
    out_shape=jax.ShapeDtypeStruct((SPLIT, 128), jnp.float32),
)


@functools.partial(
    pl.kernel,
    mesh=_mesh,
    out_type=jax.ShapeDtypeStruct((NUM_ROWS // 2, 128), jnp.float32),
    scratch_types=[
        pltpu.VMEM((RPW,), jnp.int32),
        pltpu.VMEM((RPW,), jnp.int32),
        pltpu.VMEM((RPW,), jnp.int32),
        pltpu.VMEM((2, CHUNK, 128), jnp.float32),
        pltpu.VMEM((2, CHUNK // 2, 128), jnp.float32),
        pltpu.SemaphoreType.DMA,
        pltpu.SemaphoreType.DMA,
        pltpu.SemaphoreType.DMA,
        pltpu.SemaphoreType.DMA,
    ],
    compiler_params=pltpu.CompilerParams(needs_layout_passes=False),
)
def _embed_gather(idx_hbm, pairs_hbm, out_hbm, idx_v, pidx_v, par_v,
                  pairbuf, outbuf, sg0, sg1, sw0, sw1):
    wid = lax.axis_index("s") * NC + lax.axis_index("c")
    base = wid * RPW
    pltpu.sync_copy(idx_hbm.at[pl.ds(pl.multiple_of(base, RPW), RPW)], idx_v)

    def _split(i, _):
        t = idx_v[pl.ds(i * L, L)]
        pidx_v[pl.ds(i * L, L)] = jnp.where(t < SPLIT, t, t - SPLIT)
        par_v[pl.ds(i * L, L)] = jnp.where(t < SPLIT, 0, DIM)
        return 0

    lax.fori_loop(0, RPW // L, _split, 0)

    sg = (sg0, sg1)
    sw = (sw0, sw1)

    def _fire_gather(k):
        return pltpu.async_copy(
            pairs_hbm.at[pidx_v.at[pl.ds(k * CHUNK, CHUNK)]],
            pairbuf.at[k % 2],
            sg[k % 2],
        )

    gathers = [None] * NCHUNK
    writes = [None] * NCHUNK
    gathers[0] = _fire_gather(0)
    for k in range(NCHUNK):
        if k + 1 < NCHUNK:
            gathers[k + 1] = _fire_gather(k + 1)
        gathers[k].wait()
        if k >= 2:
            writes[k - 2].wait()

        def _select(g, _):
            rid = g * L + lax.iota(jnp.int32, L)
            par = par_v[pl.ds(pl.multiple_of(k * CHUNK, L) + g * L, L)]
            pr = rid >> 1
            colbase = (rid & 1) << 6
            for c in range(DIM):
                vals = plsc.load_gather(pairbuf.at[k % 2], [rid, par + c])
                plsc.store_scatter(outbuf.at[k % 2], [pr, colbase + c], vals)
            return 0

        lax.fori_loop(0, CHUNK // L, _select, 0)
        writes[k] = pltpu.async_copy(
            outbuf.at[k % 2],
            out_hbm.at[pl.ds(wid * (RPW // 2) + k * (CHUNK // 2), CHUNK // 2)],
            sw[k % 2],
        )
    writes[NCHUNK - 2].wait()
    writes[NCHUNK - 1].wait()


def kernel(x, table):
    idx = x.reshape(NUM_ROWS).astype(jnp.int32)
    tt = table.T
    pairs = _tc_transpose(tt, tt)
    out = _embed_gather(idx, pairs)
    return out.reshape(BATCH, HIST, DIM)


# speedup vs baseline: 1.4060x; 1.4060x over previous
import functools

import jax
import jax.numpy as jnp
from jax import lax
from jax.experimental import pallas as pl
from jax.experimental.pallas import tpu as pltpu
from jax.experimental.pallas import tpu_sc as plsc

BATCH = 4096
HIST = 20
DIM = 64
NUM_ROWS = BATCH * HIST
CHUNK = 128
N_CHUNKS = NUM_ROWS // CHUNK
NC, NS = 2, 16
NW = NC * NS
CHUNKS_PER_W = N_CHUNKS // NW
WAVE = CHUNKS_PER_W // 2

_mesh = plsc.VectorSubcoreMesh(core_axis_name="c", subcore_axis_name="s")


@functools.partial(
    pl.kernel,
    mesh=_mesh,
    out_type=jax.ShapeDtypeStruct((N_CHUNKS, CHUNK, DIM), jnp.float32),
    scratch_types=[
        pltpu.VMEM((CHUNKS_PER_W, CHUNK), jnp.int32),
        pltpu.VMEM((WAVE, CHUNK, DIM), jnp.float32),
        pltpu.SemaphoreType.DMA,
    ],
    compiler_params=pltpu.CompilerParams(use_tc_tiling_on_sc=False),
)
def _embed_gather(idx_hbm, table_hbm, out_hbm, idx_v, rows_v, sem):
    wid = lax.axis_index("s") * NC + lax.axis_index("c")
    base = wid * CHUNKS_PER_W
    pltpu.sync_copy(idx_hbm.at[wid], idx_v)
    for p in range(CHUNKS_PER_W // WAVE):
        copies = [
            pltpu.async_copy(
                table_hbm.at[idx_v.at[p * WAVE + j]], rows_v.at[j], sem
            )
            for j in range(WAVE)
        ]
        for c in copies:
            c.wait()
        pltpu.sync_copy(rows_v, out_hbm.at[pl.ds(base + p * WAVE, WAVE)])


def kernel(x, table):
    idx = x.reshape(NW, CHUNKS_PER_W, CHUNK).astype(jnp.int32)
    out = _embed_gather(idx, table)
    return out.reshape(BATCH, HIST, DIM)
